# baseline (speedup 1.0000x reference)
import functools

import jax
import jax.numpy as jnp
from jax import lax
from jax.experimental import pallas as pl
from jax.experimental.pallas import tpu as pltpu
from jax.experimental.pallas import tpu_sc as plsc

N = 1024
P = N * (N - 1) // 2
NB = P // 128
NC = 2
NS = 16
NW = NC * NS
NBW = 128
CHUNK = 32
L = 16


def _pairs_body(juncs_hbm, lines_hbm, jidx_hbm, table_v, lines_v, jidx_v,
                sem_l, sem_j):
    wid = lax.axis_index("s") * NC + lax.axis_index("c")
    big = wid < 28
    b0 = jnp.where(big, wid * 128, 28 + wid * 127)
    p0 = b0 * 128
    p1 = p0 + jnp.where(big, 128 * 128, 127 * 128)

    pltpu.sync_copy(juncs_hbm, table_v)

    lane = lax.broadcasted_iota(jnp.int32, (L,), 0)

    def _off(i):
        return i * (N - 1) - lax.shift_right_logical(i * (i - 1), 1)

    def _row_of(target):
        def bisect_body(_, c):
            lo, hi = c
            mid = lax.shift_right_logical(lo + hi + 1, 1)
            pred = _off(mid) <= target
            return jnp.where(pred, mid, lo), jnp.where(pred, hi, mid - 1)

        i_row, _ = lax.fori_loop(0, 10, bisect_body,
                                 (jnp.int32(0), jnp.int32(N - 1)))
        return i_row

    i0 = _row_of(p0)
    j0 = i0 + 1 + (p0 - _off(i0))
    i_end = _row_of(p1 - 1)
    trip = lax.shift_right_logical(p1 - p0, 4) + (i_end - i0) + 2

    def _tbl_idx(n, c):
        return (lax.shift_left(lax.shift_right_logical(n, 7), 8)
                + jnp.bitwise_and(n, jnp.int32(127)) + c * 128)

    def _row_vecs(i):
        iv = jnp.full((L,), i, jnp.int32)
        xi = plsc.load_gather(table_v, [_tbl_idx(iv, 0)])
        yi = plsc.load_gather(table_v, [_tbl_idx(iv, 1)])
        return iv, xi, yi

    def body(_, c):
        p, i, j, q = c[:4]
        iv, xi, yi = c[4:]
        cnt = jnp.minimum(jnp.minimum(jnp.int32(L), N - j), p1 - p)
        msk = lane < cnt
        jv = j + lane
        jx = _tbl_idx(jv, 0)
        xj = plsc.load_gather(table_v, [jx], mask=msk)
        yj = plsc.load_gather(table_v, [jx + 128], mask=msk)
        r = q + lane
        blk = lax.shift_right_logical(r, 7)
        low = jnp.bitwise_and(r, jnp.int32(127))
        b4 = lax.shift_left(blk, 9) + low
        b2 = lax.shift_left(blk, 8) + low
        plsc.store_scatter(lines_v, [b4], xi, mask=msk)
        plsc.store_scatter(lines_v, [b4 + 128], yi, mask=msk)
        plsc.store_scatter(lines_v, [b4 + 256], xj, mask=msk)
        plsc.store_scatter(lines_v, [b4 + 384], yj, mask=msk)
        plsc.store_scatter(jidx_v, [b2], iv, mask=msk)
        plsc.store_scatter(jidx_v, [b2 + 128], jv, mask=msk)
        p = p + cnt
        qn = q + cnt
        j = j + cnt

        ck = lax.shift_right_logical(q, 12)

        @pl.when(lax.shift_right_logical(qn, 12) != ck)
        def _():
            pltpu.async_copy(
                lines_v.at[pl.ds(ck * (CHUNK * 512), CHUNK * 512)],
                lines_hbm.at[pl.ds(b0 * 512 + ck * (CHUNK * 512),
                                   CHUNK * 512)],
                sem_l)
            pltpu.async_copy(
                jidx_v.at[pl.ds(ck * (CHUNK * 256), CHUNK * 256)],
                jidx_hbm.at[pl.ds(b0 * 256 + ck * (CHUNK * 256),
                                  CHUNK * 256)],
                sem_j)

        row_done = jnp.logical_and(j >= N, p < p1)
        i = jnp.where(row_done, i + 1, i)
        j = jnp.where(row_done, i + 1, j)
        iv, xi, yi = lax.cond(row_done, _row_vecs,
                              lambda _: (iv, xi, yi), i)
        return p, i, j, qn, iv, xi, yi

    iv0, xi0, yi0 = _row_vecs(i0)
    lax.fori_loop(0, trip, body,
                  (p0, i0, j0, jnp.int32(0), iv0, xi0, yi0))

    @pl.when(big)
    def _():
        for ck in range(4):
            pltpu.make_async_copy(
                lines_v.at[pl.ds(ck * (CHUNK * 512), CHUNK * 512)],
                lines_hbm.at[pl.ds(b0 * 512 + ck * (CHUNK * 512),
                                   CHUNK * 512)],
                sem_l).wait()
            pltpu.make_async_copy(
                jidx_v.at[pl.ds(ck * (CHUNK * 256), CHUNK * 256)],
                jidx_hbm.at[pl.ds(b0 * 256 + ck * (CHUNK * 256),
                                  CHUNK * 256)],
                sem_j).wait()

    @pl.when(jnp.logical_not(big))
    def _():
        for ck in range(3):
            pltpu.make_async_copy(
                lines_v.at[pl.ds(ck * (CHUNK * 512), CHUNK * 512)],
                lines_hbm.at[pl.ds(b0 * 512 + ck * (CHUNK * 512),
                                   CHUNK * 512)],
                sem_l).wait()
            pltpu.make_async_copy(
                jidx_v.at[pl.ds(ck * (CHUNK * 256), CHUNK * 256)],
                jidx_hbm.at[pl.ds(b0 * 256 + ck * (CHUNK * 256),
                                  CHUNK * 256)],
                sem_j).wait()
        pltpu.sync_copy(
            lines_v.at[pl.ds(3 * (CHUNK * 512), 31 * 512)],
            lines_hbm.at[pl.ds(b0 * 512 + 3 * (CHUNK * 512), 31 * 512)])
        pltpu.sync_copy(
            jidx_v.at[pl.ds(3 * (CHUNK * 256), 31 * 256)],
            jidx_hbm.at[pl.ds(b0 * 256 + 3 * (CHUNK * 256), 31 * 256)])


@jax.jit
def _pairs(juncs_pred):
    mesh = plsc.VectorSubcoreMesh(core_axis_name="c", subcore_axis_name="s")
    k = functools.partial(
        pl.kernel,
        mesh=mesh,
        out_type=[
            jax.ShapeDtypeStruct((P * 4,), jnp.float32),
            jax.ShapeDtypeStruct((P * 2,), jnp.int32),
        ],
        scratch_types=[
            pltpu.VMEM((N * 2,), jnp.float32),
            pltpu.VMEM((NBW * 512,), jnp.float32),
            pltpu.VMEM((NBW * 256,), jnp.int32),
            pltpu.SemaphoreType.DMA,
            pltpu.SemaphoreType.DMA,
        ],
        compiler_params=pltpu.CompilerParams(needs_layout_passes=False),
    )(_pairs_body)
    juncs_flat = (juncs_pred.reshape(8, 128, 2)
                  .transpose(0, 2, 1).reshape(-1))
    lines_flat, jidx_flat = k(juncs_flat)
    lines_pred = (lines_flat.reshape(NB, 4, 128)
                  .transpose(0, 2, 1).reshape(P, 4))
    jidx = (jidx_flat.reshape(NB, 2, 128)
            .transpose(0, 2, 1).reshape(P, 2))
    return lines_pred, jidx


def kernel(img_idx, juncs_pred, meta):
    lines_pred, jidx = _pairs(juncs_pred)
    labels = jnp.ones((P,), dtype=jnp.int32)
    label_scores = jnp.ones((P,), dtype=jnp.float32)
    scores = jnp.broadcast_to(jnp.array([0.0, 1.0], dtype=jnp.float32),
                              (P, 2))
    return (lines_pred, labels, label_scores, jidx, scores)


# speedup vs baseline: 1.0579x; 1.0579x over previous
import functools

import jax
import jax.numpy as jnp
from jax import lax
from jax.experimental import pallas as pl
from jax.experimental.pallas import tpu as pltpu
from jax.experimental.pallas import tpu_sc as plsc

N = 1024
P = N * (N - 1) // 2
NB = P // 128
NC = 2
NS = 16
NW = NC * NS
NBW = 128
CHUNK = 32
L = 16


def _pairs_body(juncs_hbm, lines_hbm, jidx_hbm, table_v, lines_v, jidx_v,
                sem_l, sem_j):
    wid = lax.axis_index("s") * NC + lax.axis_index("c")
    big = wid < 28
    b0 = jnp.where(big, wid * 128, 28 + wid * 127)
    p0 = b0 * 128
    p1 = p0 + jnp.where(big, 128 * 128, 127 * 128)

    pltpu.sync_copy(juncs_hbm, table_v)

    lane = lax.broadcasted_iota(jnp.int32, (L,), 0)

    def _off(i):
        return i * (N - 1) - lax.shift_right_logical(i * (i - 1), 1)

    def _row_of(target):
        def bisect_body(_, c):
            lo, hi = c
            mid = lax.shift_right_logical(lo + hi + 1, 1)
            pred = _off(mid) <= target
            return jnp.where(pred, mid, lo), jnp.where(pred, hi, mid - 1)

        i_row, _ = lax.fori_loop(0, 10, bisect_body,
                                 (jnp.int32(0), jnp.int32(N - 1)))
        return i_row

    i0 = _row_of(p0)
    j0 = i0 + 1 + (p0 - _off(i0))

    def _tbl_idx(n, c):
        return (lax.shift_left(lax.shift_right_logical(n, 7), 8)
                + jnp.bitwise_and(n, jnp.int32(127)) + c * 128)

    def _row_vecs(i):
        iv = jnp.full((L,), i, jnp.int32)
        xi = plsc.load_gather(table_v, [_tbl_idx(iv, 0)])
        yi = plsc.load_gather(table_v, [_tbl_idx(iv, 1)])
        return iv, xi, yi

    def _make_body(chunk_end):
        def body(_, c):
            p, i, j, q = c[:4]
            iv, xi, yi = c[4:]
            cnt = jnp.minimum(jnp.minimum(jnp.int32(L), N - j),
                              chunk_end - p)
            msk = lane < cnt
            jv = j + lane
            jx = _tbl_idx(jv, 0)
            xj = plsc.load_gather(table_v, [jx], mask=msk)
            yj = plsc.load_gather(table_v, [jx + 128], mask=msk)
            r = q + lane
            blk = lax.shift_right_logical(r, 7)
            low = jnp.bitwise_and(r, jnp.int32(127))
            b4 = lax.shift_left(blk, 9) + low
            b2 = lax.shift_left(blk, 8) + low
            plsc.store_scatter(lines_v, [b4], xi, mask=msk)
            plsc.store_scatter(lines_v, [b4 + 128], yi, mask=msk)
            plsc.store_scatter(lines_v, [b4 + 256], xj, mask=msk)
            plsc.store_scatter(lines_v, [b4 + 384], yj, mask=msk)
            plsc.store_scatter(jidx_v, [b2], iv, mask=msk)
            plsc.store_scatter(jidx_v, [b2 + 128], jv, mask=msk)
            p = p + cnt
            q = q + cnt
            j = j + cnt
            row_done = jnp.logical_and(j >= N, i < jnp.int32(N - 2))
            i = jnp.where(row_done, i + 1, i)
            j = jnp.where(row_done, i + 1, j)
            iv, xi, yi = lax.cond(row_done, _row_vecs,
                                  lambda _: (iv, xi, yi), i)
            return p, i, j, q, iv, xi, yi

        return body

    iv0, xi0, yi0 = _row_vecs(i0)
    carry = (p0, i0, j0, jnp.int32(0), iv0, xi0, yi0)
    for ck in range(4):
        chunk_end = jnp.minimum(p1, p0 + (ck + 1) * (CHUNK * 128))
        i_ce = _row_of(chunk_end - 1)
        trip = (lax.shift_right_logical(chunk_end - carry[0], 4)
                + (i_ce - carry[1]) + 2)
        carry = lax.fori_loop(0, trip, _make_body(chunk_end), carry)
        if ck < 3:
            pltpu.async_copy(
                lines_v.at[pl.ds(ck * (CHUNK * 512), CHUNK * 512)],
                lines_hbm.at[pl.ds(b0 * 512 + ck * (CHUNK * 512),
                                   CHUNK * 512)],
                sem_l)
            pltpu.async_copy(
                jidx_v.at[pl.ds(ck * (CHUNK * 256), CHUNK * 256)],
                jidx_hbm.at[pl.ds(b0 * 256 + ck * (CHUNK * 256),
                                  CHUNK * 256)],
                sem_j)

    @pl.when(big)
    def _():
        pltpu.async_copy(
            lines_v.at[pl.ds(3 * (CHUNK * 512), CHUNK * 512)],
            lines_hbm.at[pl.ds(b0 * 512 + 3 * (CHUNK * 512),
                               CHUNK * 512)],
            sem_l)
        pltpu.async_copy(
            jidx_v.at[pl.ds(3 * (CHUNK * 256), CHUNK * 256)],
            jidx_hbm.at[pl.ds(b0 * 256 + 3 * (CHUNK * 256),
                              CHUNK * 256)],
            sem_j)

    @pl.when(big)
    def _():
        for ck in range(4):
            pltpu.make_async_copy(
                lines_v.at[pl.ds(ck * (CHUNK * 512), CHUNK * 512)],
                lines_hbm.at[pl.ds(b0 * 512 + ck * (CHUNK * 512),
                                   CHUNK * 512)],
                sem_l).wait()
            pltpu.make_async_copy(
                jidx_v.at[pl.ds(ck * (CHUNK * 256), CHUNK * 256)],
                jidx_hbm.at[pl.ds(b0 * 256 + ck * (CHUNK * 256),
                                  CHUNK * 256)],
                sem_j).wait()

    @pl.when(jnp.logical_not(big))
    def _():
        for ck in range(3):
            pltpu.make_async_copy(
                lines_v.at[pl.ds(ck * (CHUNK * 512), CHUNK * 512)],
                lines_hbm.at[pl.ds(b0 * 512 + ck * (CHUNK * 512),
                                   CHUNK * 512)],
                sem_l).wait()
            pltpu.make_async_copy(
                jidx_v.at[pl.ds(ck * (CHUNK * 256), CHUNK * 256)],
                jidx_hbm.at[pl.ds(b0 * 256 + ck * (CHUNK * 256),
                                  CHUNK * 256)],
                sem_j).wait()
        pltpu.sync_copy(
            lines_v.at[pl.ds(3 * (CHUNK * 512), 31 * 512)],
            lines_hbm.at[pl.ds(b0 * 512 + 3 * (CHUNK * 512), 31 * 512)])
        pltpu.sync_copy(
            jidx_v.at[pl.ds(3 * (CHUNK * 256), 31 * 256)],
            jidx_hbm.at[pl.ds(b0 * 256 + 3 * (CHUNK * 256), 31 * 256)])


@jax.jit
def _pairs(juncs_pred):
    mesh = plsc.VectorSubcoreMesh(core_axis_name="c", subcore_axis_name="s")
    k = functools.partial(
        pl.kernel,
        mesh=mesh,
        out_type=[
            jax.ShapeDtypeStruct((P * 4,), jnp.float32),
            jax.ShapeDtypeStruct((P * 2,), jnp.int32),
        ],
        scratch_types=[
            pltpu.VMEM((N * 2,), jnp.float32),
            pltpu.VMEM((NBW * 512,), jnp.float32),
            pltpu.VMEM((NBW * 256,), jnp.int32),
            pltpu.SemaphoreType.DMA,
            pltpu.SemaphoreType.DMA,
        ],
        compiler_params=pltpu.CompilerParams(needs_layout_passes=False),
    )(_pairs_body)
    juncs_flat = (juncs_pred.reshape(8, 128, 2)
                  .transpose(0, 2, 1).reshape(-1))
    lines_flat, jidx_flat = k(juncs_flat)
    lines_pred = (lines_flat.reshape(NB, 4, 128)
                  .transpose(0, 2, 1).reshape(P, 4))
    jidx = (jidx_flat.reshape(NB, 2, 128)
            .transpose(0, 2, 1).reshape(P, 2))
    return lines_pred, jidx


def kernel(img_idx, juncs_pred, meta):
    lines_pred, jidx = _pairs(juncs_pred)
    labels = jnp.ones((P,), dtype=jnp.int32)
    label_scores = jnp.ones((P,), dtype=jnp.float32)
    scores = jnp.broadcast_to(jnp.array([0.0, 1.0], dtype=jnp.float32),
                              (P, 2))
    return (lines_pred, labels, label_scores, jidx, scores)


# speedup vs baseline: 1.1795x; 1.1150x over previous
import functools

import jax
import jax.numpy as jnp
from jax import lax
from jax.experimental import pallas as pl
from jax.experimental.pallas import tpu as pltpu
from jax.experimental.pallas import tpu_sc as plsc

N = 1024
P = N * (N - 1) // 2
NB = P // 128
NC = 2
NS = 16
NW = NC * NS
NBW = 128
CHUNK = 32
L = 16
PB = 8192


def _pairs_body(juncs_hbm, lines_hbm, jidx_hbm, labels_hbm, lsc_hbm,
                scores_hbm, table_v, lines_v, jidx_v, ones_i_v, ones_f_v,
                spat_v, sem_l, sem_j, sem_c):
    wid = lax.axis_index("s") * NC + lax.axis_index("c")
    big = wid < 28
    b0 = jnp.where(big, wid * 128, 28 + wid * 127)
    p0 = b0 * 128
    p1 = p0 + jnp.where(big, 128 * 128, 127 * 128)

    pltpu.sync_copy(juncs_hbm, table_v)

    lane = lax.broadcasted_iota(jnp.int32, (L,), 0)

    ones_i = jnp.ones((L,), jnp.int32)
    ones_f = jnp.ones((L,), jnp.float32)

    def fill_body(k, _):
        base = k * L
        ones_i_v[pl.ds(base, L)] = ones_i
        ones_f_v[pl.ds(base, L)] = ones_f
        half = jnp.bitwise_and(lax.shift_right_logical(k, 3),
                               jnp.int32(1))
        spat_v[pl.ds(base, L)] = jnp.full((L,), half.astype(jnp.float32))
        return 0

    lax.fori_loop(0, PB // L, fill_body, 0)

    def _const_copies():
        big_list = []
        for t in range(2):
            big_list.append((ones_i_v,
                             labels_hbm.at[pl.ds(p0 + t * PB, PB)]))
            big_list.append((ones_f_v,
                             lsc_hbm.at[pl.ds(p0 + t * PB, PB)]))
        for t in range(4):
            big_list.append((spat_v,
                             scores_hbm.at[pl.ds(b0 * 256 + t * PB, PB)]))
        sm_list = [
            (ones_i_v, labels_hbm.at[pl.ds(p0, PB)]),
            (ones_i_v.at[pl.ds(0, 8064)],
             labels_hbm.at[pl.ds(p0 + PB, 8064)]),
            (ones_f_v, lsc_hbm.at[pl.ds(p0, PB)]),
            (ones_f_v.at[pl.ds(0, 8064)],
             lsc_hbm.at[pl.ds(p0 + PB, 8064)]),
        ]
        for t in range(3):
            sm_list.append((spat_v,
                            scores_hbm.at[pl.ds(b0 * 256 + t * PB, PB)]))
        sm_list.append((spat_v.at[pl.ds(0, 7936)],
                        scores_hbm.at[pl.ds(b0 * 256 + 3 * PB, 7936)]))
        return big_list, sm_list

    big_list, sm_list = _const_copies()

    @pl.when(big)
    def _():
        for src, dst in big_list:
            pltpu.async_copy(src, dst, sem_c)

    @pl.when(jnp.logical_not(big))
    def _():
        for src, dst in sm_list:
            pltpu.async_copy(src, dst, sem_c)

    def _off(i):
        return i * (N - 1) - lax.shift_right_logical(i * (i - 1), 1)

    def _row_of(target):
        def bisect_body(_, c):
            lo, hi = c
            mid = lax.shift_right_logical(lo + hi + 1, 1)
            pred = _off(mid) <= target
            return jnp.where(pred, mid, lo), jnp.where(pred, hi, mid - 1)

        i_row, _ = lax.fori_loop(0, 10, bisect_body,
                                 (jnp.int32(0), jnp.int32(N - 1)))
        return i_row

    i0 = _row_of(p0)
    j0 = i0 + 1 + (p0 - _off(i0))

    def _tbl_idx(n, c):
        return (lax.shift_left(lax.shift_right_logical(n, 7), 8)
                + jnp.bitwise_and(n, jnp.int32(127)) + c * 128)

    def _row_vecs(i):
        iv = jnp.full((L,), i, jnp.int32)
        xi = plsc.load_gather(table_v, [_tbl_idx(iv, 0)])
        yi = plsc.load_gather(table_v, [_tbl_idx(iv, 1)])
        return iv, xi, yi

    def _make_body(chunk_end):
        def body(_, c):
            p, i, j, q = c[:4]
            iv, xi, yi = c[4:]
            cnt = jnp.minimum(jnp.minimum(jnp.int32(L), N - j),
                              chunk_end - p)
            msk = lane < cnt
            jv = j + lane
            jx = _tbl_idx(jv, 0)
            xj = plsc.load_gather(table_v, [jx], mask=msk)
            yj = plsc.load_gather(table_v, [jx + 128], mask=msk)
            r = q + lane
            blk = lax.shift_right_logical(r, 7)
            low = jnp.bitwise_and(r, jnp.int32(127))
            b4 = lax.shift_left(blk, 9) + low
            b2 = lax.shift_left(blk, 8) + low
            plsc.store_scatter(lines_v, [b4], xi, mask=msk)
            plsc.store_scatter(lines_v, [b4 + 128], yi, mask=msk)
            plsc.store_scatter(lines_v, [b4 + 256], xj, mask=msk)
            plsc.store_scatter(lines_v, [b4 + 384], yj, mask=msk)
            plsc.store_scatter(jidx_v, [b2], iv, mask=msk)
            plsc.store_scatter(jidx_v, [b2 + 128], jv, mask=msk)
            p = p + cnt
            q = q + cnt
            j = j + cnt
            row_done = jnp.logical_and(j >= N, i < jnp.int32(N - 2))
            i = jnp.where(row_done, i + 1, i)
            j = jnp.where(row_done, i + 1, j)
            iv, xi, yi = lax.cond(row_done, _row_vecs,
                                  lambda _: (iv, xi, yi), i)
            return p, i, j, q, iv, xi, yi

        return body

    iv0, xi0, yi0 = _row_vecs(i0)
    carry = (p0, i0, j0, jnp.int32(0), iv0, xi0, yi0)
    for ck in range(4):
        chunk_end = jnp.minimum(p1, p0 + (ck + 1) * (CHUNK * 128))
        i_ce = _row_of(chunk_end - 1)
        trip = (lax.shift_right_logical(chunk_end - carry[0], 4)
                + (i_ce - carry[1]) + 2)
        carry = lax.fori_loop(0, trip, _make_body(chunk_end), carry)
        if ck < 3:
            pltpu.async_copy(
                lines_v.at[pl.ds(ck * (CHUNK * 512), CHUNK * 512)],
                lines_hbm.at[pl.ds(b0 * 512 + ck * (CHUNK * 512),
                                   CHUNK * 512)],
                sem_l)
            pltpu.async_copy(
                jidx_v.at[pl.ds(ck * (CHUNK * 256), CHUNK * 256)],
                jidx_hbm.at[pl.ds(b0 * 256 + ck * (CHUNK * 256),
                                  CHUNK * 256)],
                sem_j)

    @pl.when(big)
    def _():
        pltpu.async_copy(
            lines_v.at[pl.ds(3 * (CHUNK * 512), CHUNK * 512)],
            lines_hbm.at[pl.ds(b0 * 512 + 3 * (CHUNK * 512),
                               CHUNK * 512)],
            sem_l)
        pltpu.async_copy(
            jidx_v.at[pl.ds(3 * (CHUNK * 256), CHUNK * 256)],
            jidx_hbm.at[pl.ds(b0 * 256 + 3 * (CHUNK * 256),
                              CHUNK * 256)],
            sem_j)

    @pl.when(big)
    def _():
        for ck in range(4):
            pltpu.make_async_copy(
                lines_v.at[pl.ds(ck * (CHUNK * 512), CHUNK * 512)],
                lines_hbm.at[pl.ds(b0 * 512 + ck * (CHUNK * 512),
                                   CHUNK * 512)],
                sem_l).wait()
            pltpu.make_async_copy(
                jidx_v.at[pl.ds(ck * (CHUNK * 256), CHUNK * 256)],
                jidx_hbm.at[pl.ds(b0 * 256 + ck * (CHUNK * 256),
                                  CHUNK * 256)],
                sem_j).wait()
        for src, dst in big_list:
            pltpu.make_async_copy(src, dst, sem_c).wait()

    @pl.when(jnp.logical_not(big))
    def _():
        for ck in range(3):
            pltpu.make_async_copy(
                lines_v.at[pl.ds(ck * (CHUNK * 512), CHUNK * 512)],
                lines_hbm.at[pl.ds(b0 * 512 + ck * (CHUNK * 512),
                                   CHUNK * 512)],
                sem_l).wait()
            pltpu.make_async_copy(
                jidx_v.at[pl.ds(ck * (CHUNK * 256), CHUNK * 256)],
                jidx_hbm.at[pl.ds(b0 * 256 + ck * (CHUNK * 256),
                                  CHUNK * 256)],
                sem_j).wait()
        pltpu.sync_copy(
            lines_v.at[pl.ds(3 * (CHUNK * 512), 31 * 512)],
            lines_hbm.at[pl.ds(b0 * 512 + 3 * (CHUNK * 512), 31 * 512)])
        pltpu.sync_copy(
            jidx_v.at[pl.ds(3 * (CHUNK * 256), 31 * 256)],
            jidx_hbm.at[pl.ds(b0 * 256 + 3 * (CHUNK * 256), 31 * 256)])
        for src, dst in sm_list:
            pltpu.make_async_copy(src, dst, sem_c).wait()


@jax.jit
def _pairs(juncs_pred):
    mesh = plsc.VectorSubcoreMesh(core_axis_name="c", subcore_axis_name="s")
    k = functools.partial(
        pl.kernel,
        mesh=mesh,
        out_type=[
            jax.ShapeDtypeStruct((P * 4,), jnp.float32),
            jax.ShapeDtypeStruct((P * 2,), jnp.int32),
            jax.ShapeDtypeStruct((P,), jnp.int32),
            jax.ShapeDtypeStruct((P,), jnp.float32),
            jax.ShapeDtypeStruct((P * 2,), jnp.float32),
        ],
        scratch_types=[
            pltpu.VMEM((N * 2,), jnp.float32),
            pltpu.VMEM((NBW * 512,), jnp.float32),
            pltpu.VMEM((NBW * 256,), jnp.int32),
            pltpu.VMEM((PB,), jnp.int32),
            pltpu.VMEM((PB,), jnp.float32),
            pltpu.VMEM((PB,), jnp.float32),
            pltpu.SemaphoreType.DMA,
            pltpu.SemaphoreType.DMA,
            pltpu.SemaphoreType.DMA,
        ],
        compiler_params=pltpu.CompilerParams(needs_layout_passes=False),
    )(_pairs_body)
    juncs_flat = (juncs_pred.reshape(8, 128, 2)
                  .transpose(0, 2, 1).reshape(-1))
    lines_flat, jidx_flat, labels, label_scores, scores_flat = \
        k(juncs_flat)
    lines_pred = (lines_flat.reshape(NB, 4, 128)
                  .transpose(0, 2, 1).reshape(P, 4))
    jidx = (jidx_flat.reshape(NB, 2, 128)
            .transpose(0, 2, 1).reshape(P, 2))
    scores = (scores_flat.reshape(NB, 2, 128)
              .transpose(0, 2, 1).reshape(P, 2))
    return lines_pred, jidx, labels, label_scores, scores


def kernel(img_idx, juncs_pred, meta):
    lines_pred, jidx, labels, label_scores, scores = _pairs(juncs_pred)
    return (lines_pred, labels, label_scores, jidx, scores)
